# baseline (speedup 1.0000x reference)
import jax
import jax.numpy as jnp
from jax.experimental import pallas as pl
from jax.experimental.pallas import tpu as pltpu

_H = 48
_W = 48
_A = 9
_N = _A * _H * _W
_ROWS = _N // 128
_K = 1000
_KP = 1024
_POST = 300
_OUTP = 304
_THRESH = 0.7
_MIN_SIZE = 2.0

_SZ_W = (4.0, 8.0, 16.0, 4.0, 8.0, 8.0, 16.0, 16.0, 32.0)
_SZ_H = (4.0, 8.0, 16.0, 8.0, 4.0, 16.0, 8.0, 32.0, 16.0)


def _box_kernel(sc_ref, dx_ref, dy_ref, dw_ref, dh_ref,
                x1_ref, y1_ref, w_ref, h_ref, x2n_ref, y2n_ref, ar_ref, fs_ref):
    b = (jax.lax.broadcasted_iota(jnp.int32, (_ROWS, 128), 0) * 128
         + jax.lax.broadcasted_iota(jnp.int32, (_ROWS, 128), 1))
    a = b // (_H * _W)
    rem = b - a * (_H * _W)
    yi = rem // _W
    xi = rem - yi * _W

    wsz = jnp.full((_ROWS, 128), _SZ_W[0], jnp.float32)
    hsz = jnp.full((_ROWS, 128), _SZ_H[0], jnp.float32)
    for k in range(1, _A):
        m = a == k
        wsz = jnp.where(m, _SZ_W[k], wsz)
        hsz = jnp.where(m, _SZ_H[k], hsz)

    anc_x = xi.astype(jnp.float32) + 0.5 - wsz / 2.0
    anc_y = yi.astype(jnp.float32) + 0.5 - hsz / 2.0

    ax = jnp.maximum(anc_x + dx_ref[...], 0.0)
    ay = jnp.maximum(anc_y + dy_ref[...], 0.0)
    aw = jnp.maximum(wsz + dw_ref[...], 0.0)
    ah = jnp.maximum(hsz + dh_ref[...], 0.0)

    x2r = ax + aw
    y2r = ay + ah
    x1 = jnp.minimum(ax, float(_H))
    y1 = jnp.minimum(ay, float(_W))
    x2 = jnp.minimum(x2r, float(_H))
    y2 = jnp.minimum(y2r, float(_W))
    w = x2 - x1
    h = y2 - y1

    keep = (w >= _MIN_SIZE) & (h >= _MIN_SIZE)
    x1_ref[...] = x1
    y1_ref[...] = y1
    w_ref[...] = w
    h_ref[...] = h
    x2n_ref[...] = x1 + w
    y2n_ref[...] = y1 + h
    ar_ref[...] = w * h
    fs_ref[...] = jnp.where(keep, sc_ref[...], -jnp.inf)


def _lane_prefix_incl(x):
    for s in (1, 2, 4, 8, 16, 32, 64):
        x = x + jnp.concatenate(
            [jnp.zeros((1, s), jnp.float32), x[:, :128 - s]], axis=1)
    return x


def _select_kernel(fs_ref, attr_ref, comp_ref):
    comp_ref[...] = jnp.zeros((_KP, 9), jnp.float32)
    ik_all = jax.lax.bitcast_convert_type(fs_ref[...], jnp.int32)

    def bs_body(_, lohi):
        lo, hi = lohi
        mid = lo + (hi - lo + 1) // 2
        cnt = jnp.sum(jnp.where(ik_all >= mid, 1.0, 0.0))
        ge = cnt >= float(_K)
        return (jnp.where(ge, mid, lo), jnp.where(ge, hi, mid - 1))

    lo0 = jnp.int32(-8388608)
    hi0 = jnp.int32(0x3F800000)
    theta, _ = jax.lax.fori_loop(0, 31, bs_body, (lo0, hi0))

    n_gt = jnp.sum(jnp.where(ik_all > theta, 1.0, 0.0))
    need_eq = float(_K) - n_gt
    diota = jax.lax.broadcasted_iota(jnp.int32, (_KP, 1), 0)

    def row_body(r, carry):
        base, eqb = carry
        ik = jax.lax.bitcast_convert_type(fs_ref[pl.ds(r, 1), :], jnp.int32)
        gt = ik > theta
        eq = ik == theta
        eqf = jnp.where(eq, 1.0, 0.0)
        excl_eq = _lane_prefix_incl(eqf) - eqf
        member = gt | (eq & ((eqb + excl_eq) < need_eq))
        mf = jnp.where(member, 1.0, 0.0)
        excl_m = _lane_prefix_incl(mf) - mf
        dst = (base + excl_m).astype(jnp.int32)
        oh = jnp.where((diota == dst) & member, 1.0, 0.0)
        a = attr_ref[pl.ds(r * 128, 128), :]
        comp_ref[...] += jnp.dot(oh, a, preferred_element_type=jnp.float32,
                                 precision=jax.lax.Precision.HIGHEST)
        return (base + jnp.sum(mf), eqb + jnp.sum(eqf))

    jax.lax.fori_loop(0, _ROWS, row_body, (0.0, 0.0))

    slot = jax.lax.broadcasted_iota(jnp.int32, (_KP, 9), 0)
    col = jax.lax.broadcasted_iota(jnp.int32, (_KP, 9), 1)
    cur = comp_ref[...]
    padm = slot >= _K
    cur = jnp.where(padm & (col == 5), -1e30, cur)
    cur = jnp.where(padm & (col == 3), -1.0, cur)
    comp_ref[...] = cur


def _nms_kernel(rows_ref, cols_ref, pprop_ref, out_ref, ov_scr):
    cx1 = cols_ref[0:1, :]
    cy1 = cols_ref[1:2, :]
    cx2 = cols_ref[2:3, :]
    cy2 = cols_ref[3:4, :]
    car = cols_ref[4:5, :]
    csc = cols_ref[5:6, :]

    for bi in range(_KP // 128):
        blk = rows_ref[bi * 128:(bi + 1) * 128, :]
        rx1 = blk[:, 0:1]
        ry1 = blk[:, 1:2]
        rx2 = blk[:, 2:3]
        ry2 = blk[:, 3:4]
        xx1 = jnp.maximum(rx1, cx1)
        yy1 = jnp.maximum(ry1, cy1)
        xx2 = jnp.minimum(rx2, cx2)
        yy2 = jnp.minimum(ry2, cy2)
        wm = jnp.maximum(xx2 - xx1 + 1.0, 0.0)
        hm = jnp.maximum(yy2 - yy1 + 1.0, 0.0)
        ov_scr[bi * 128:(bi + 1) * 128, :] = (wm * hm) / jnp.maximum(car, 1e-6)

    kmf0 = jnp.where(csc >= 0.0, 1.0, 0.0)
    lane = jax.lax.broadcasted_iota(jnp.int32, (1, _KP), 1)

    def body(i, kmf):
        row = ov_scr[pl.ds(i, 1), :]
        alive = jnp.sum(jnp.where(lane == i, kmf, 0.0)) > 0.0
        supf = jnp.where((row >= _THRESH) & alive & (lane != i), 1.0, 0.0)
        return kmf * (1.0 - supf)

    kmf = jax.lax.fori_loop(0, _KP, body, kmf0)

    c = kmf
    s = 1
    while s < _KP:
        c = c + jnp.concatenate(
            [jnp.zeros((1, s), jnp.float32), c[:, :_KP - s]], axis=1)
        s *= 2
    rank = c - 1.0
    sel = (kmf > 0.0) & (rank < float(_POST))

    kidx = jax.lax.broadcasted_iota(jnp.int32, (_OUTP, _KP), 0)
    onehot = jnp.where((kidx == rank.astype(jnp.int32)) & sel, 1.0, 0.0)
    out_ref[...] = jnp.dot(onehot, pprop_ref[...],
                           preferred_element_type=jnp.float32,
                           precision=jax.lax.Precision.HIGHEST)


def kernel(scores, bbox_deltas, image_metadata):
    f32 = jnp.float32
    sc = scores.reshape(_ROWS, 128)
    d = bbox_deltas.reshape(_A, _H, _W, 4)
    dx = d[..., 0].reshape(_ROWS, 128)
    dy = d[..., 1].reshape(_ROWS, 128)
    dw = d[..., 2].reshape(_ROWS, 128)
    dh = d[..., 3].reshape(_ROWS, 128)

    shp = jax.ShapeDtypeStruct((_ROWS, 128), f32)
    x1, y1, w, h, x2n, y2n, ar, fs = pl.pallas_call(
        _box_kernel,
        out_shape=[shp] * 8,
    )(sc, dx, dy, dw, dh)

    idxf = jnp.arange(_N, dtype=f32).reshape(_ROWS, 128)
    fs_fin = jnp.maximum(fs, -1e30)
    attr = jnp.stack([x1, y1, x2n, y2n, ar, fs_fin, w, h, idxf],
                     axis=-1).reshape(_N, 9)

    comp = pl.pallas_call(
        _select_kernel,
        out_shape=jax.ShapeDtypeStruct((_KP, 9), f32),
    )(fs, attr)

    _, _, _, take = jax.lax.sort(
        (-comp[:, 3], comp[:, 5], -comp[:, 8],
         jnp.arange(_KP, dtype=jnp.int32)),
        num_keys=3)
    srt = comp[take][:, 0:8]
    rows = srt
    cols = rows.T
    pprop = jnp.concatenate([rows[:, 0:2], rows[:, 6:8]], axis=1)

    out = pl.pallas_call(
        _nms_kernel,
        out_shape=jax.ShapeDtypeStruct((_OUTP, 4), f32),
        scratch_shapes=[pltpu.VMEM((_KP, _KP), f32)],
    )(rows, cols, pprop)

    return out[:_POST][None]


# speedup vs baseline: 1.9914x; 1.9914x over previous
import jax
import jax.numpy as jnp
from jax.experimental import pallas as pl
from jax.experimental.pallas import tpu as pltpu

_H = 48
_W = 48
_A = 9
_N = _A * _H * _W
_ROWS = _N // 128
_K = 1000
_KP = 1024
_POST = 300
_OUTP = 304
_THRESH = 0.7
_MIN_SIZE = 2.0

_SZ_W = (4.0, 8.0, 16.0, 4.0, 8.0, 8.0, 16.0, 16.0, 32.0)
_SZ_H = (4.0, 8.0, 16.0, 8.0, 4.0, 16.0, 8.0, 32.0, 16.0)


def _box_kernel(sc_ref, dx_ref, dy_ref, dw_ref, dh_ref,
                x1_ref, y1_ref, w_ref, h_ref, x2n_ref, y2n_ref, ar_ref, fs_ref):
    b = (jax.lax.broadcasted_iota(jnp.int32, (_ROWS, 128), 0) * 128
         + jax.lax.broadcasted_iota(jnp.int32, (_ROWS, 128), 1))
    a = b // (_H * _W)
    rem = b - a * (_H * _W)
    yi = rem // _W
    xi = rem - yi * _W

    wsz = jnp.full((_ROWS, 128), _SZ_W[0], jnp.float32)
    hsz = jnp.full((_ROWS, 128), _SZ_H[0], jnp.float32)
    for k in range(1, _A):
        m = a == k
        wsz = jnp.where(m, _SZ_W[k], wsz)
        hsz = jnp.where(m, _SZ_H[k], hsz)

    anc_x = xi.astype(jnp.float32) + 0.5 - wsz / 2.0
    anc_y = yi.astype(jnp.float32) + 0.5 - hsz / 2.0

    ax = jnp.maximum(anc_x + dx_ref[...], 0.0)
    ay = jnp.maximum(anc_y + dy_ref[...], 0.0)
    aw = jnp.maximum(wsz + dw_ref[...], 0.0)
    ah = jnp.maximum(hsz + dh_ref[...], 0.0)

    x2r = ax + aw
    y2r = ay + ah
    x1 = jnp.minimum(ax, float(_H))
    y1 = jnp.minimum(ay, float(_W))
    x2 = jnp.minimum(x2r, float(_H))
    y2 = jnp.minimum(y2r, float(_W))
    w = x2 - x1
    h = y2 - y1

    keep = (w >= _MIN_SIZE) & (h >= _MIN_SIZE)
    x1_ref[...] = x1
    y1_ref[...] = y1
    w_ref[...] = w
    h_ref[...] = h
    x2n_ref[...] = x1 + w
    y2n_ref[...] = y1 + h
    ar_ref[...] = w * h
    fs_ref[...] = jnp.where(keep, sc_ref[...], -jnp.inf)


def _nms_kernel(rows_ref, cols_ref, pprop_ref, out_ref, ov_scr):
    cx1 = cols_ref[0:1, :]
    cy1 = cols_ref[1:2, :]
    cx2 = cols_ref[2:3, :]
    cy2 = cols_ref[3:4, :]
    car = cols_ref[4:5, :]
    csc = cols_ref[5:6, :]

    for bi in range(_KP // 128):
        blk = rows_ref[bi * 128:(bi + 1) * 128, :]
        rx1 = blk[:, 0:1]
        ry1 = blk[:, 1:2]
        rx2 = blk[:, 2:3]
        ry2 = blk[:, 3:4]
        xx1 = jnp.maximum(rx1, cx1)
        yy1 = jnp.maximum(ry1, cy1)
        xx2 = jnp.minimum(rx2, cx2)
        yy2 = jnp.minimum(ry2, cy2)
        wm = jnp.maximum(xx2 - xx1 + 1.0, 0.0)
        hm = jnp.maximum(yy2 - yy1 + 1.0, 0.0)
        ov_scr[bi * 128:(bi + 1) * 128, :] = (wm * hm) / jnp.maximum(car, 1e-6)

    kmf0 = jnp.where(csc >= 0.0, 1.0, 0.0)
    lane = jax.lax.broadcasted_iota(jnp.int32, (1, _KP), 1)

    def body(i, kmf):
        row = ov_scr[pl.ds(i, 1), :]
        alive = jnp.sum(jnp.where(lane == i, kmf, 0.0)) > 0.0
        supf = jnp.where((row >= _THRESH) & alive & (lane != i), 1.0, 0.0)
        return kmf * (1.0 - supf)

    kmf = jax.lax.fori_loop(0, _KP, body, kmf0)

    c = kmf
    s = 1
    while s < _KP:
        c = c + jnp.concatenate(
            [jnp.zeros((1, s), jnp.float32), c[:, :_KP - s]], axis=1)
        s *= 2
    rank = c - 1.0
    sel = (kmf > 0.0) & (rank < float(_POST))

    kidx = jax.lax.broadcasted_iota(jnp.int32, (_OUTP, _KP), 0)
    onehot = jnp.where((kidx == rank.astype(jnp.int32)) & sel, 1.0, 0.0)
    out_ref[...] = jnp.dot(onehot, pprop_ref[...],
                           preferred_element_type=jnp.float32,
                           precision=jax.lax.Precision.HIGHEST)


def kernel(scores, bbox_deltas, image_metadata):
    f32 = jnp.float32
    sc = scores.reshape(_ROWS, 128)
    d = bbox_deltas.reshape(_A, _H, _W, 4)
    dx = d[..., 0].reshape(_ROWS, 128)
    dy = d[..., 1].reshape(_ROWS, 128)
    dw = d[..., 2].reshape(_ROWS, 128)
    dh = d[..., 3].reshape(_ROWS, 128)

    shp = jax.ShapeDtypeStruct((_ROWS, 128), f32)
    x1, y1, w, h, x2n, y2n, ar, fs = pl.pallas_call(
        _box_kernel,
        out_shape=[shp] * 8,
    )(sc, dx, dy, dw, dh)

    idxf = jnp.arange(_N, dtype=f32).reshape(_ROWS, 128)
    fs_fin = jnp.maximum(fs, -1e30)
    attr = jnp.stack([x1, y1, x2n, y2n, ar, fs_fin, w, h, idxf],
                     axis=-1).reshape(_N, 9)

    _, order = jax.lax.top_k(fs.reshape(-1), _K)
    members = jnp.sort(order)
    comp = attr[members]
    padrow = jnp.zeros((_KP - _K, 9), f32).at[:, 5].set(-1e30).at[:, 3].set(-1.0)
    comp = jnp.concatenate([comp, padrow], axis=0)

    _, _, _, take = jax.lax.sort(
        (-comp[:, 3], comp[:, 5], -comp[:, 8],
         jnp.arange(_KP, dtype=jnp.int32)),
        num_keys=3)
    srt = comp[take][:, 0:8]
    rows = srt
    cols = rows.T
    pprop = jnp.concatenate([rows[:, 0:2], rows[:, 6:8]], axis=1)

    out = pl.pallas_call(
        _nms_kernel,
        out_shape=jax.ShapeDtypeStruct((_OUTP, 4), f32),
        scratch_shapes=[pltpu.VMEM((_KP, _KP), f32)],
    )(rows, cols, pprop)

    return out[:_POST][None]


# speedup vs baseline: 2.0190x; 1.0139x over previous
import jax
import jax.numpy as jnp
from jax.experimental import pallas as pl
from jax.experimental.pallas import tpu as pltpu

_H = 48
_W = 48
_A = 9
_N = _A * _H * _W
_ROWS = _N // 128
_K = 1000
_KP = 1024
_POST = 300
_OUTP = 304
_THRESH = 0.7
_MIN_SIZE = 2.0

_SZ_W = (4.0, 8.0, 16.0, 4.0, 8.0, 8.0, 16.0, 16.0, 32.0)
_SZ_H = (4.0, 8.0, 16.0, 8.0, 4.0, 16.0, 8.0, 32.0, 16.0)


def _box_kernel(sc_ref, dx_ref, dy_ref, dw_ref, dh_ref,
                x1_ref, y1_ref, w_ref, h_ref, x2n_ref, y2n_ref, ar_ref, fs_ref):
    b = (jax.lax.broadcasted_iota(jnp.int32, (_ROWS, 128), 0) * 128
         + jax.lax.broadcasted_iota(jnp.int32, (_ROWS, 128), 1))
    a = b // (_H * _W)
    rem = b - a * (_H * _W)
    yi = rem // _W
    xi = rem - yi * _W

    wsz = jnp.full((_ROWS, 128), _SZ_W[0], jnp.float32)
    hsz = jnp.full((_ROWS, 128), _SZ_H[0], jnp.float32)
    for k in range(1, _A):
        m = a == k
        wsz = jnp.where(m, _SZ_W[k], wsz)
        hsz = jnp.where(m, _SZ_H[k], hsz)

    anc_x = xi.astype(jnp.float32) + 0.5 - wsz / 2.0
    anc_y = yi.astype(jnp.float32) + 0.5 - hsz / 2.0

    ax = jnp.maximum(anc_x + dx_ref[...], 0.0)
    ay = jnp.maximum(anc_y + dy_ref[...], 0.0)
    aw = jnp.maximum(wsz + dw_ref[...], 0.0)
    ah = jnp.maximum(hsz + dh_ref[...], 0.0)

    x2r = ax + aw
    y2r = ay + ah
    x1 = jnp.minimum(ax, float(_H))
    y1 = jnp.minimum(ay, float(_W))
    x2 = jnp.minimum(x2r, float(_H))
    y2 = jnp.minimum(y2r, float(_W))
    w = x2 - x1
    h = y2 - y1

    keep = (w >= _MIN_SIZE) & (h >= _MIN_SIZE)
    x1_ref[...] = x1
    y1_ref[...] = y1
    w_ref[...] = w
    h_ref[...] = h
    x2n_ref[...] = x1 + w
    y2n_ref[...] = y1 + h
    ar_ref[...] = w * h
    fs_ref[...] = jnp.where(keep, sc_ref[...], -jnp.inf)


def _nms_kernel(rows_ref, cols_ref, pprop_ref, out_ref, ov_scr):
    cx1 = cols_ref[0:1, :]
    cy1 = cols_ref[1:2, :]
    cx2 = cols_ref[2:3, :]
    cy2 = cols_ref[3:4, :]
    car = cols_ref[4:5, :]
    csc = cols_ref[5:6, :]

    for bi in range(_KP // 128):
        blk = rows_ref[bi * 128:(bi + 1) * 128, :]
        rx1 = blk[:, 0:1]
        ry1 = blk[:, 1:2]
        rx2 = blk[:, 2:3]
        ry2 = blk[:, 3:4]
        xx1 = jnp.maximum(rx1, cx1)
        yy1 = jnp.maximum(ry1, cy1)
        xx2 = jnp.minimum(rx2, cx2)
        yy2 = jnp.minimum(ry2, cy2)
        wm = jnp.maximum(xx2 - xx1 + 1.0, 0.0)
        hm = jnp.maximum(yy2 - yy1 + 1.0, 0.0)
        ov_scr[bi * 128:(bi + 1) * 128, :] = (wm * hm) / jnp.maximum(car, 1e-6)

    kmf0 = jnp.where(csc >= 0.0, 1.0, 0.0)
    lane = jax.lax.broadcasted_iota(jnp.int32, (1, _KP), 1)

    def body(i, kmf):
        row = ov_scr[pl.ds(i, 1), :]
        alive = jnp.sum(jnp.where(lane == i, kmf, 0.0)) > 0.0
        supf = jnp.where((row >= _THRESH) & alive & (lane != i), 1.0, 0.0)
        return kmf * (1.0 - supf)

    kmf = jax.lax.fori_loop(0, _K, body, kmf0)

    c = kmf
    s = 1
    while s < _KP:
        c = c + jnp.concatenate(
            [jnp.zeros((1, s), jnp.float32), c[:, :_KP - s]], axis=1)
        s *= 2
    rank = c - 1.0
    sel = (kmf > 0.0) & (rank < float(_POST))

    kidx = jax.lax.broadcasted_iota(jnp.int32, (_OUTP, _KP), 0)
    onehot = jnp.where((kidx == rank.astype(jnp.int32)) & sel, 1.0, 0.0)
    out_ref[...] = jnp.dot(onehot, pprop_ref[...],
                           preferred_element_type=jnp.float32,
                           precision=jax.lax.Precision.HIGHEST)


def kernel(scores, bbox_deltas, image_metadata):
    f32 = jnp.float32
    sc = scores.reshape(_ROWS, 128)
    d = bbox_deltas.reshape(_A, _H, _W, 4)
    dx = d[..., 0].reshape(_ROWS, 128)
    dy = d[..., 1].reshape(_ROWS, 128)
    dw = d[..., 2].reshape(_ROWS, 128)
    dh = d[..., 3].reshape(_ROWS, 128)

    shp = jax.ShapeDtypeStruct((_ROWS, 128), f32)
    x1, y1, w, h, x2n, y2n, ar, fs = pl.pallas_call(
        _box_kernel,
        out_shape=[shp] * 8,
    )(sc, dx, dy, dw, dh)

    idxf = jnp.arange(_N, dtype=f32).reshape(_ROWS, 128)
    fs_fin = jnp.maximum(fs, -1e30)
    attr = jnp.stack([x1, y1, x2n, y2n, ar, fs_fin, w, h, idxf],
                     axis=-1).reshape(_N, 9)

    _, order = jax.lax.top_k(fs.reshape(-1), _K)
    members = jnp.sort(order)
    comp = attr[members]
    padrow = jnp.zeros((_KP - _K, 9), f32).at[:, 5].set(-1e30).at[:, 3].set(-1.0)
    comp = jnp.concatenate([comp, padrow], axis=0)

    _, _, _, take = jax.lax.sort(
        (-comp[:, 3], comp[:, 5], -comp[:, 8],
         jnp.arange(_KP, dtype=jnp.int32)),
        num_keys=3)
    srt = comp[take][:, 0:8]
    rows = srt
    cols = rows.T
    pprop = jnp.concatenate([rows[:, 0:2], rows[:, 6:8]], axis=1)

    out = pl.pallas_call(
        _nms_kernel,
        out_shape=jax.ShapeDtypeStruct((_OUTP, 4), f32),
        scratch_shapes=[pltpu.VMEM((_KP, _KP), f32)],
    )(rows, cols, pprop)

    return out[:_POST][None]
